# baseline (speedup 1.0000x reference)
import functools

import jax
import jax.numpy as jnp
from jax import lax
from jax.experimental import pallas as pl
from jax.experimental.pallas import tpu as pltpu
from jax.experimental.pallas import tpu_sc as plsc

NC = 2
NS = 16
NW = NC * NS
CHUNK = 128

def _sc_mesh():
    return plsc.VectorSubcoreMesh(core_axis_name="c", subcore_axis_name="s")


def _pad_rows(n):
    return (n // 2048 + 1) * 2048


def _sc_deg_body(dst_hbm, zeros_hbm, out_hbm, dst_v, ones_v, acc, k_chunks,
                 n_pad):
    cid = lax.axis_index("c")
    sid = lax.axis_index("s")
    wid = sid * NC + cid
    rows = n_pad // NS
    base = pl.multiple_of(sid * rows, 128)
    out_base = pl.multiple_of(cid * n_pad + sid * rows, 128)
    pltpu.sync_copy(zeros_hbm.at[pl.ds(base, rows)], acc.at[pl.ds(base, rows)])
    pltpu.sync_copy(dst_hbm.at[wid], dst_v)
    for l in range(CHUNK // 16):
        ones_v[pl.ds(l * 16, 16)] = jnp.ones((16,), jnp.float32)
    plsc.subcore_barrier()

    def body(j, carry):
        pltpu.sync_copy(ones_v, acc.at[dst_v.at[j]], add=True)
        return carry

    lax.fori_loop(0, k_chunks, body, 0)
    plsc.subcore_barrier()
    pltpu.sync_copy(acc.at[pl.ds(base, rows)],
                    out_hbm.at[pl.ds(out_base, rows)])


DEPTH = 2


def _sc_agg_body(y_hbm, src_hbm, dst_hbm, zeros_hbm, out_hbm,
                 sb0, sb1, db0, db1, db2, db3, r0, r1, acc,
                 ss0, ss1, sd0, sd1, sd2, sd3, g0, g1, c0, c1,
                 k_chunks, n_pad, d):
    sbuf = (sb0, sb1)
    dbuf = (db0, db1, db2, db3)
    rows_v = (r0, r1)
    ssem = (ss0, ss1)
    dsem = (sd0, sd1, sd2, sd3)
    gsem = (g0, g1)
    csem = (c0, c1)
    cid = lax.axis_index("c")
    sid = lax.axis_index("s")
    wid = sid * NC + cid
    rows = n_pad // NS
    base = pl.multiple_of(sid * rows, 128)
    pltpu.sync_copy(zeros_hbm.at[pl.ds(base, rows)], acc.at[pl.ds(base, rows)])
    plsc.subcore_barrier()

    def fire_sidx(j, b):
        pltpu.async_copy(src_hbm.at[wid, j], sbuf[b], ssem[b])

    def fire_didx(j, q):
        pltpu.async_copy(dst_hbm.at[wid, j], dbuf[q], dsem[q])

    def fire_gather(j, b):
        pltpu.make_async_copy(src_hbm.at[wid, j], sbuf[b], ssem[b]).wait()
        pltpu.async_copy(y_hbm.at[sbuf[b]], rows_v[b], gsem[b])

    def scat_wait(b, q):
        pltpu.make_async_copy(rows_v[b], acc.at[dbuf[q]], csem[b]).wait()

    fire_didx(0, 0)
    fire_didx(1, 1)
    fire_didx(2, 2)
    fire_sidx(0, 0)
    fire_sidx(1, 1)
    fire_gather(0, 0)

    def body(jj, carry):
        for i in range(4):
            j = jj * 4 + i
            b = i % 2
            b2 = 1 - b
            q = i
            pltpu.make_async_copy(y_hbm.at[sbuf[b]], rows_v[b],
                                  gsem[b]).wait()

            @pl.when(j >= 1)
            def _():
                scat_wait(b2, (i + 3) % 4)

            @pl.when(j + 1 < k_chunks)
            def _():
                fire_gather(j + 1, b2)

            @pl.when(j + 3 < k_chunks)
            def _():
                fire_didx(j + 3, (i + 3) % 4)

            @pl.when(j + 2 < k_chunks)
            def _():
                fire_sidx(j + 2, b)

            pltpu.make_async_copy(dst_hbm.at[wid, j], dbuf[q],
                                  dsem[q]).wait()
            pltpu.async_copy(rows_v[b], acc.at[dbuf[q]], csem[b], add=True)
        return carry

    lax.fori_loop(0, k_chunks // 4, body, 0)
    scat_wait((k_chunks - 1) % 2, (k_chunks - 1) % 4)
    plsc.subcore_barrier()
    pltpu.sync_copy(acc.at[pl.ds(base, rows)],
                    out_hbm.at[cid, pl.ds(base, rows)])


def _tc_lin0_body(x_ref, w_ref, d0_ref, d1_ref, y_ref, dinv_ref):
    deg = d0_ref[...] + d1_ref[...] + 1.0
    dinv = lax.rsqrt(deg)
    dinv_ref[...] = dinv
    y_ref[...] = jnp.dot(x_ref[...], w_ref[...],
                         preferred_element_type=jnp.float32) * dinv


def _tc_mid_body(s0_ref, s1_ref, y_ref, dinv_ref, b_ref, w_ref, y2_ref):
    dinv = dinv_ref[...]
    h = jnp.maximum(
        dinv * (s0_ref[0] + s1_ref[0] + y_ref[...]) + b_ref[...], 0.0)
    y2_ref[...] = jnp.dot(h, w_ref[...],
                          preferred_element_type=jnp.float32) * dinv


def _tc_out_body(s0_ref, s1_ref, y_ref, dinv_ref, b_ref, o_ref):
    o_ref[...] = jnp.maximum(
        dinv_ref[...] * (s0_ref[0] + s1_ref[0] + y_ref[...])
        + b_ref[...], 0.0)


def kernel(x, edge_index, W0, b0, W1, b1):
    n, d_in = x.shape
    d_hid = W0.shape[1]
    e = edge_index.shape[1]
    n_pad = _pad_rows(n)
    k_chunks = -(-e // (NW * CHUNK))
    k_chunks = max(4, -(-k_chunks // 4) * 4)
    e_pad = k_chunks * NW * CHUNK
    pad = e_pad - e

    src = edge_index[0].astype(jnp.int32)
    dst = edge_index[1].astype(jnp.int32)
    if pad:
        fill = jnp.arange(pad, dtype=jnp.int32)
        src = jnp.concatenate([src, fill % n])
        dst = jnp.concatenate([dst, n + fill % (n_pad - n)])
    src_slab = src.reshape(NW, k_chunks, CHUNK)
    dst_slab = dst.reshape(NW, k_chunks, CHUNK)
    zeros1 = jnp.zeros((n_pad,), jnp.float32)
    zeros2 = jnp.zeros((n_pad, d_hid), jnp.float32)

    deg_fn = pl.kernel(
        functools.partial(_sc_deg_body, k_chunks=k_chunks, n_pad=n_pad),
        out_type=jax.ShapeDtypeStruct((NC * n_pad,), jnp.float32),
        mesh=_sc_mesh(),
        scratch_types=[
            pltpu.VMEM((k_chunks, CHUNK), jnp.int32),
            pltpu.VMEM((CHUNK,), jnp.float32),
            pltpu.VMEM_SHARED((n_pad,), jnp.float32),
        ],
    )
    degp = deg_fn(dst_slab, zeros1).reshape(NC, n_pad)

    agg_fn = pl.kernel(
        functools.partial(_sc_agg_body, k_chunks=k_chunks, n_pad=n_pad,
                          d=d_hid),
        out_type=jax.ShapeDtypeStruct((NC, n_pad, d_hid), jnp.float32),
        mesh=_sc_mesh(),
        scratch_types=[pltpu.VMEM((CHUNK,), jnp.int32)] * 6
        + [pltpu.VMEM((CHUNK, d_hid), jnp.float32)] * DEPTH + [
            pltpu.VMEM_SHARED((n_pad, d_hid), jnp.float32),
        ] + [pltpu.SemaphoreType.DMA] * 10,
    )

    bm = 512
    grid = (-(-n // bm),)
    row_spec = pl.BlockSpec((bm, d_hid), lambda i: (i, 0))
    col_spec = pl.BlockSpec((bm, 1), lambda i: (i, 0))
    w_spec = pl.BlockSpec((d_in, d_hid), lambda i: (0, 0))
    b_spec = pl.BlockSpec((1, d_hid), lambda i: (0, 0))

    d0 = degp[0, :n, None]
    d1 = degp[1, :n, None]
    y1, dinv = pl.pallas_call(
        _tc_lin0_body,
        grid=grid,
        in_specs=[pl.BlockSpec((bm, d_in), lambda i: (i, 0)), w_spec,
                  col_spec, col_spec],
        out_specs=[row_spec, col_spec],
        out_shape=[jax.ShapeDtypeStruct((n, d_hid), jnp.float32),
                   jax.ShapeDtypeStruct((n, 1), jnp.float32)],
    )(x, W0, d0, d1)

    s_spec0 = pl.BlockSpec((1, bm, d_hid), lambda i: (0, i, 0))
    s_spec1 = pl.BlockSpec((1, bm, d_hid), lambda i: (1, i, 0))

    s1 = agg_fn(y1, src_slab, dst_slab, zeros2)
    y2 = pl.pallas_call(
        _tc_mid_body,
        grid=grid,
        in_specs=[s_spec0, s_spec1, row_spec, col_spec, b_spec, w_spec],
        out_specs=row_spec,
        out_shape=jax.ShapeDtypeStruct((n, d_hid), jnp.float32),
    )(s1, s1, y1, dinv, b0[None, :], W1)

    s2 = agg_fn(y2, src_slab, dst_slab, zeros2)
    out = pl.pallas_call(
        _tc_out_body,
        grid=grid,
        in_specs=[s_spec0, s_spec1, row_spec, col_spec, b_spec],
        out_specs=row_spec,
        out_shape=jax.ShapeDtypeStruct((n, d_hid), jnp.float32),
    )(s2, s2, y2, dinv, b1[None, :])
    return out


# speedup vs baseline: 1.0228x; 1.0228x over previous
import functools

import jax
import jax.numpy as jnp
from jax import lax
from jax.experimental import pallas as pl
from jax.experimental.pallas import tpu as pltpu
from jax.experimental.pallas import tpu_sc as plsc

NC = 2
NS = 16
NW = NC * NS
CHUNK = 64

def _sc_mesh():
    return plsc.VectorSubcoreMesh(core_axis_name="c", subcore_axis_name="s")


def _pad_rows(n):
    return (n // 2048 + 1) * 2048


def _sc_deg_body(dst_hbm, zeros_hbm, out_hbm, dst_v, ones_v, acc, k_chunks,
                 n_pad):
    cid = lax.axis_index("c")
    sid = lax.axis_index("s")
    wid = sid * NC + cid
    rows = n_pad // NS
    base = pl.multiple_of(sid * rows, 128)
    out_base = pl.multiple_of(cid * n_pad + sid * rows, 128)
    pltpu.sync_copy(zeros_hbm.at[pl.ds(base, rows)], acc.at[pl.ds(base, rows)])
    pltpu.sync_copy(dst_hbm.at[wid], dst_v)
    for l in range(CHUNK // 16):
        ones_v[pl.ds(l * 16, 16)] = jnp.ones((16,), jnp.float32)
    plsc.subcore_barrier()

    def body(j, carry):
        pltpu.sync_copy(ones_v, acc.at[dst_v.at[j]], add=True)
        return carry

    lax.fori_loop(0, k_chunks, body, 0)
    plsc.subcore_barrier()
    pltpu.sync_copy(acc.at[pl.ds(base, rows)],
                    out_hbm.at[pl.ds(out_base, rows)])


DEPTH = 2


def _sc_agg_body(y_hbm, src_hbm, dst_hbm, zeros_hbm, out_hbm,
                 sb0, sb1, sb2, sb3, db0, db1, db2, db3,
                 r0, r1, r2, r3, acc,
                 ss0, ss1, ss2, ss3, sd0, sd1, sd2, sd3,
                 g0, g1, g2, g3, c0, c1, c2, c3,
                 k_chunks, n_pad, d):
    sbuf = (sb0, sb1, sb2, sb3)
    dbuf = (db0, db1, db2, db3)
    rows_v = (r0, r1, r2, r3)
    ssem = (ss0, ss1, ss2, ss3)
    dsem = (sd0, sd1, sd2, sd3)
    gsem = (g0, g1, g2, g3)
    csem = (c0, c1, c2, c3)
    cid = lax.axis_index("c")
    sid = lax.axis_index("s")
    wid = sid * NC + cid
    rows = n_pad // NS
    base = pl.multiple_of(sid * rows, 8)
    pltpu.sync_copy(zeros_hbm.at[pl.ds(base, rows)], acc.at[pl.ds(base, rows)])
    plsc.subcore_barrier()

    def fire_sidx(j, q):
        pltpu.async_copy(src_hbm.at[wid, j], sbuf[q], ssem[q])

    def fire_didx(j, q):
        pltpu.async_copy(dst_hbm.at[wid, j], dbuf[q], dsem[q])

    def fire_gather(j, q):
        pltpu.make_async_copy(src_hbm.at[wid, j], sbuf[q], ssem[q]).wait()
        pltpu.async_copy(y_hbm.at[sbuf[q]], rows_v[q], gsem[q])

    def scat_wait(q):
        pltpu.make_async_copy(rows_v[q], acc.at[dbuf[q]], csem[q]).wait()

    for q in range(4):
        fire_sidx(q, q)
    fire_didx(0, 0)
    fire_didx(1, 1)
    fire_gather(0, 0)
    fire_gather(1, 1)

    def body(jj, carry):
        for i in range(4):
            j = jj * 4 + i
            s2 = (i + 2) % 4
            pltpu.make_async_copy(y_hbm.at[sbuf[i]], rows_v[i],
                                  gsem[i]).wait()

            @pl.when(j >= 2)
            def _():
                scat_wait(s2)

            @pl.when(j + 2 < k_chunks)
            def _():
                fire_gather(j + 2, s2)
                fire_didx(j + 2, s2)

            @pl.when(j + 4 < k_chunks)
            def _():
                fire_sidx(j + 4, i)

            pltpu.make_async_copy(dst_hbm.at[wid, j], dbuf[i],
                                  dsem[i]).wait()
            pltpu.async_copy(rows_v[i], acc.at[dbuf[i]], csem[i], add=True)
        return carry

    lax.fori_loop(0, k_chunks // 4, body, 0)
    scat_wait((k_chunks - 2) % 4)
    scat_wait((k_chunks - 1) % 4)
    plsc.subcore_barrier()
    pltpu.sync_copy(acc.at[pl.ds(base, rows)],
                    out_hbm.at[cid, pl.ds(base, rows)])


def _tc_lin0_body(x_ref, w_ref, d0_ref, d1_ref, y_ref, dinv_ref):
    deg = d0_ref[...] + d1_ref[...] + 1.0
    dinv = lax.rsqrt(deg)
    dinv_ref[...] = dinv
    y_ref[...] = jnp.dot(x_ref[...], w_ref[...],
                         preferred_element_type=jnp.float32) * dinv


def _tc_mid_body(s0_ref, s1_ref, y_ref, dinv_ref, b_ref, w_ref, y2_ref):
    dinv = dinv_ref[...]
    h = jnp.maximum(
        dinv * (s0_ref[0] + s1_ref[0] + y_ref[...]) + b_ref[...], 0.0)
    y2_ref[...] = jnp.dot(h, w_ref[...],
                          preferred_element_type=jnp.float32) * dinv


def _tc_out_body(s0_ref, s1_ref, y_ref, dinv_ref, b_ref, o_ref):
    o_ref[...] = jnp.maximum(
        dinv_ref[...] * (s0_ref[0] + s1_ref[0] + y_ref[...])
        + b_ref[...], 0.0)


def kernel(x, edge_index, W0, b0, W1, b1):
    n, d_in = x.shape
    d_hid = W0.shape[1]
    e = edge_index.shape[1]
    n_pad = _pad_rows(n)
    n_pad_a = (n // 128 + 1) * 128
    k_chunks = -(-e // (NW * CHUNK))
    k_chunks = max(4, -(-k_chunks // 4) * 4)
    e_pad = k_chunks * NW * CHUNK
    pad = e_pad - e

    src = edge_index[0].astype(jnp.int32)
    dst = edge_index[1].astype(jnp.int32)
    if pad:
        fill = jnp.arange(pad, dtype=jnp.int32)
        src = jnp.concatenate([src, fill % n])
        dst = jnp.concatenate([dst, n + fill % (n_pad_a - n)])
    src_slab = src.reshape(NW, k_chunks, CHUNK)
    dst_slab = dst.reshape(NW, k_chunks, CHUNK)
    zeros1 = jnp.zeros((n_pad,), jnp.float32)
    zeros2 = jnp.zeros((n_pad_a, d_hid), jnp.float32)

    deg_fn = pl.kernel(
        functools.partial(_sc_deg_body, k_chunks=k_chunks, n_pad=n_pad),
        out_type=jax.ShapeDtypeStruct((NC * n_pad,), jnp.float32),
        mesh=_sc_mesh(),
        scratch_types=[
            pltpu.VMEM((k_chunks, CHUNK), jnp.int32),
            pltpu.VMEM((CHUNK,), jnp.float32),
            pltpu.VMEM_SHARED((n_pad,), jnp.float32),
        ],
    )
    degp = deg_fn(dst_slab, zeros1).reshape(NC, n_pad)

    agg_fn = pl.kernel(
        functools.partial(_sc_agg_body, k_chunks=k_chunks, n_pad=n_pad_a,
                          d=d_hid),
        out_type=jax.ShapeDtypeStruct((NC, n_pad_a, d_hid), jnp.float32),
        mesh=_sc_mesh(),
        scratch_types=[pltpu.VMEM((CHUNK,), jnp.int32)] * 8
        + [pltpu.VMEM((CHUNK, d_hid), jnp.float32)] * 4 + [
            pltpu.VMEM_SHARED((n_pad_a, d_hid), jnp.float32),
        ] + [pltpu.SemaphoreType.DMA] * 16,
    )

    bm = 512
    grid = (-(-n // bm),)
    row_spec = pl.BlockSpec((bm, d_hid), lambda i: (i, 0))
    col_spec = pl.BlockSpec((bm, 1), lambda i: (i, 0))
    w_spec = pl.BlockSpec((d_in, d_hid), lambda i: (0, 0))
    b_spec = pl.BlockSpec((1, d_hid), lambda i: (0, 0))

    d0 = degp[0, :n, None]
    d1 = degp[1, :n, None]
    y1, dinv = pl.pallas_call(
        _tc_lin0_body,
        grid=grid,
        in_specs=[pl.BlockSpec((bm, d_in), lambda i: (i, 0)), w_spec,
                  col_spec, col_spec],
        out_specs=[row_spec, col_spec],
        out_shape=[jax.ShapeDtypeStruct((n, d_hid), jnp.float32),
                   jax.ShapeDtypeStruct((n, 1), jnp.float32)],
    )(x, W0, d0, d1)

    s_spec0 = pl.BlockSpec((1, bm, d_hid), lambda i: (0, i, 0))
    s_spec1 = pl.BlockSpec((1, bm, d_hid), lambda i: (1, i, 0))

    s1 = agg_fn(y1, src_slab, dst_slab, zeros2)
    y2 = pl.pallas_call(
        _tc_mid_body,
        grid=grid,
        in_specs=[s_spec0, s_spec1, row_spec, col_spec, b_spec, w_spec],
        out_specs=row_spec,
        out_shape=jax.ShapeDtypeStruct((n, d_hid), jnp.float32),
    )(s1, s1, y1, dinv, b0[None, :], W1)

    s2 = agg_fn(y2, src_slab, dst_slab, zeros2)
    out = pl.pallas_call(
        _tc_out_body,
        grid=grid,
        in_specs=[s_spec0, s_spec1, row_spec, col_spec, b_spec],
        out_specs=row_spec,
        out_shape=jax.ShapeDtypeStruct((n, d_hid), jnp.float32),
    )(s2, s2, y2, dinv, b1[None, :])
    return out


# speedup vs baseline: 1.0236x; 1.0007x over previous
import functools

import jax
import jax.numpy as jnp
from jax import lax
from jax.experimental import pallas as pl
from jax.experimental.pallas import tpu as pltpu
from jax.experimental.pallas import tpu_sc as plsc

NC = 2
NS = 16
NW = NC * NS
CHUNK = 64

def _sc_mesh():
    return plsc.VectorSubcoreMesh(core_axis_name="c", subcore_axis_name="s")


def _pad_rows(n):
    return (n // 2048 + 1) * 2048


def _sc_deg_body(dst_hbm, zeros_hbm, out_hbm, dst_v, ones_v, acc, k_chunks,
                 n_pad):
    cid = lax.axis_index("c")
    sid = lax.axis_index("s")
    wid = sid * NC + cid
    rows = n_pad // NS
    base = pl.multiple_of(sid * rows, 128)
    out_base = pl.multiple_of(cid * n_pad + sid * rows, 128)
    pltpu.sync_copy(zeros_hbm.at[pl.ds(base, rows)], acc.at[pl.ds(base, rows)])
    pltpu.sync_copy(dst_hbm.at[wid], dst_v)
    for l in range(CHUNK // 16):
        ones_v[pl.ds(l * 16, 16)] = jnp.ones((16,), jnp.float32)
    plsc.subcore_barrier()

    def body(j, carry):
        pltpu.sync_copy(ones_v, acc.at[dst_v.at[j]], add=True)
        return carry

    lax.fori_loop(0, k_chunks, body, 0)
    plsc.subcore_barrier()
    pltpu.sync_copy(acc.at[pl.ds(base, rows)],
                    out_hbm.at[pl.ds(out_base, rows)])


DEPTH = 2


def _sc_agg_body(y_hbm, src_hbm, dst_hbm, zeros_hbm, out_hbm,
                 sb0, sb1, sb2, sb3, db0, db1, db2, db3,
                 r0, r1, r2, r3, acc,
                 ss0, ss1, ss2, ss3, sd0, sd1, sd2, sd3,
                 g0, g1, g2, g3, c0, c1, c2, c3,
                 k_chunks, n_pad, d):
    sbuf = (sb0, sb1, sb2, sb3)
    dbuf = (db0, db1, db2, db3)
    rows_v = (r0, r1, r2, r3)
    ssem = (ss0, ss1, ss2, ss3)
    dsem = (sd0, sd1, sd2, sd3)
    gsem = (g0, g1, g2, g3)
    csem = (c0, c1, c2, c3)
    cid = lax.axis_index("c")
    sid = lax.axis_index("s")
    wid = sid * NC + cid
    rows = n_pad // NS
    base = pl.multiple_of(sid * rows, 8)
    pltpu.sync_copy(zeros_hbm.at[pl.ds(base, rows)], acc.at[pl.ds(base, rows)])
    plsc.subcore_barrier()

    def fire_sidx(j, q):
        pltpu.async_copy(src_hbm.at[wid, j], sbuf[q], ssem[q])

    def fire_didx(j, q):
        pltpu.async_copy(dst_hbm.at[wid, j], dbuf[q], dsem[q])

    def fire_gather(j, q):
        pltpu.make_async_copy(src_hbm.at[wid, j], sbuf[q], ssem[q]).wait()
        pltpu.async_copy(y_hbm.at[sbuf[q]], rows_v[q], gsem[q])

    def scat_wait(q):
        pltpu.make_async_copy(rows_v[q], acc.at[dbuf[q]], csem[q]).wait()

    for q in range(4):
        fire_sidx(q, q)
    fire_didx(0, 0)
    fire_didx(1, 1)
    fire_gather(0, 0)
    fire_gather(1, 1)

    def body(jj, carry):
        for i in range(4):
            j = jj * 4 + i
            s2 = (i + 2) % 4
            pltpu.make_async_copy(y_hbm.at[sbuf[i]], rows_v[i],
                                  gsem[i]).wait()

            @pl.when(j >= 2)
            def _():
                scat_wait(s2)

            @pl.when(j + 2 < k_chunks)
            def _():
                fire_gather(j + 2, s2)
                fire_didx(j + 2, s2)

            @pl.when(j + 4 < k_chunks)
            def _():
                fire_sidx(j + 4, i)

            pltpu.make_async_copy(dst_hbm.at[wid, j], dbuf[i],
                                  dsem[i]).wait()
            pltpu.async_copy(rows_v[i], acc.at[dbuf[i]], csem[i], add=True)
        return carry

    lax.fori_loop(0, k_chunks // 4, body, 0)
    scat_wait((k_chunks - 2) % 4)
    scat_wait((k_chunks - 1) % 4)
    plsc.subcore_barrier()
    pltpu.sync_copy(acc.at[pl.ds(base, rows)],
                    out_hbm.at[cid, pl.ds(base, rows)])


def _tc_mm_body(x_ref, w_ref, y_ref):
    y_ref[...] = jnp.dot(x_ref[...], w_ref[...],
                         preferred_element_type=jnp.float32)


def _tc_scale_body(xw_ref, d0_ref, d1_ref, y_ref, dinv_ref):
    deg = d0_ref[...] + d1_ref[...] + 1.0
    dinv = lax.rsqrt(deg)
    dinv_ref[...] = dinv
    y_ref[...] = xw_ref[...] * dinv


def _tc_mid_body(s0_ref, s1_ref, y_ref, dinv_ref, b_ref, w_ref, y2_ref):
    dinv = dinv_ref[...]
    h = jnp.maximum(
        dinv * (s0_ref[0] + s1_ref[0] + y_ref[...]) + b_ref[...], 0.0)
    y2_ref[...] = jnp.dot(h, w_ref[...],
                          preferred_element_type=jnp.float32) * dinv


def _tc_out_body(s0_ref, s1_ref, y_ref, dinv_ref, b_ref, o_ref):
    o_ref[...] = jnp.maximum(
        dinv_ref[...] * (s0_ref[0] + s1_ref[0] + y_ref[...])
        + b_ref[...], 0.0)


def kernel(x, edge_index, W0, b0, W1, b1):
    n, d_in = x.shape
    d_hid = W0.shape[1]
    e = edge_index.shape[1]
    n_pad = _pad_rows(n)
    n_pad_a = (n // 128 + 1) * 128
    k_chunks = -(-e // (NW * CHUNK))
    k_chunks = max(4, -(-k_chunks // 4) * 4)
    e_pad = k_chunks * NW * CHUNK
    pad = e_pad - e

    src = edge_index[0].astype(jnp.int32)
    dst = edge_index[1].astype(jnp.int32)
    if pad:
        fill = jnp.arange(pad, dtype=jnp.int32)
        src = jnp.concatenate([src, fill % n])
        dst = jnp.concatenate([dst, n + fill % (n_pad_a - n)])
    src_slab = src.reshape(NW, k_chunks, CHUNK)
    dst_slab = dst.reshape(NW, k_chunks, CHUNK)
    zeros1 = jnp.zeros((n_pad,), jnp.float32)
    zeros2 = jnp.zeros((n_pad_a, d_hid), jnp.float32)

    deg_fn = pl.kernel(
        functools.partial(_sc_deg_body, k_chunks=k_chunks, n_pad=n_pad),
        out_type=jax.ShapeDtypeStruct((NC * n_pad,), jnp.float32),
        mesh=_sc_mesh(),
        scratch_types=[
            pltpu.VMEM((k_chunks, CHUNK), jnp.int32),
            pltpu.VMEM((CHUNK,), jnp.float32),
            pltpu.VMEM_SHARED((n_pad,), jnp.float32),
        ],
    )
    degp = deg_fn(dst_slab, zeros1).reshape(NC, n_pad)

    agg_fn = pl.kernel(
        functools.partial(_sc_agg_body, k_chunks=k_chunks, n_pad=n_pad_a,
                          d=d_hid),
        out_type=jax.ShapeDtypeStruct((NC, n_pad_a, d_hid), jnp.float32),
        mesh=_sc_mesh(),
        scratch_types=[pltpu.VMEM((CHUNK,), jnp.int32)] * 8
        + [pltpu.VMEM((CHUNK, d_hid), jnp.float32)] * 4 + [
            pltpu.VMEM_SHARED((n_pad_a, d_hid), jnp.float32),
        ] + [pltpu.SemaphoreType.DMA] * 16,
    )

    bm = 512
    grid = (-(-n // bm),)
    row_spec = pl.BlockSpec((bm, d_hid), lambda i: (i, 0))
    col_spec = pl.BlockSpec((bm, 1), lambda i: (i, 0))
    w_spec = pl.BlockSpec((d_in, d_hid), lambda i: (0, 0))
    b_spec = pl.BlockSpec((1, d_hid), lambda i: (0, 0))

    xw1 = pl.pallas_call(
        _tc_mm_body,
        grid=grid,
        in_specs=[pl.BlockSpec((bm, d_in), lambda i: (i, 0)), w_spec],
        out_specs=row_spec,
        out_shape=jax.ShapeDtypeStruct((n, d_hid), jnp.float32),
    )(x, W0)

    d0 = degp[0, :n, None]
    d1 = degp[1, :n, None]
    y1, dinv = pl.pallas_call(
        _tc_scale_body,
        grid=grid,
        in_specs=[row_spec, col_spec, col_spec],
        out_specs=[row_spec, col_spec],
        out_shape=[jax.ShapeDtypeStruct((n, d_hid), jnp.float32),
                   jax.ShapeDtypeStruct((n, 1), jnp.float32)],
    )(xw1, d0, d1)

    s_spec0 = pl.BlockSpec((1, bm, d_hid), lambda i: (0, i, 0))
    s_spec1 = pl.BlockSpec((1, bm, d_hid), lambda i: (1, i, 0))

    s1 = agg_fn(y1, src_slab, dst_slab, zeros2)
    y2 = pl.pallas_call(
        _tc_mid_body,
        grid=grid,
        in_specs=[s_spec0, s_spec1, row_spec, col_spec, b_spec, w_spec],
        out_specs=row_spec,
        out_shape=jax.ShapeDtypeStruct((n, d_hid), jnp.float32),
    )(s1, s1, y1, dinv, b0[None, :], W1)

    s2 = agg_fn(y2, src_slab, dst_slab, zeros2)
    out = pl.pallas_call(
        _tc_out_body,
        grid=grid,
        in_specs=[s_spec0, s_spec1, row_spec, col_spec, b_spec],
        out_specs=row_spec,
        out_shape=jax.ShapeDtypeStruct((n, d_hid), jnp.float32),
    )(s2, s2, y2, dinv, b1[None, :])
    return out


# speedup vs baseline: 1.0743x; 1.0496x over previous
import functools

import numpy as np

import jax
import jax.numpy as jnp
from jax import lax
from jax.experimental import pallas as pl
from jax.experimental.pallas import tpu as pltpu
from jax.experimental.pallas import tpu_sc as plsc

NC = 2
NS = 16
NW = NC * NS
CHUNK = 64

def _sc_mesh():
    return plsc.VectorSubcoreMesh(core_axis_name="c", subcore_axis_name="s")


def _pad_rows(n):
    return (n // 2048 + 1) * 2048


def _sc_deg_body(dst_hbm, zeros_hbm, out_hbm, dst_v, ones_v, acc, k_chunks,
                 n_pad):
    cid = lax.axis_index("c")
    sid = lax.axis_index("s")
    wid = sid * NC + cid
    rows = n_pad // NS
    base = pl.multiple_of(sid * rows, 128)
    out_base = pl.multiple_of(cid * n_pad + sid * rows, 128)
    pltpu.sync_copy(zeros_hbm.at[pl.ds(base, rows)], acc.at[pl.ds(base, rows)])
    pltpu.sync_copy(dst_hbm.at[wid], dst_v)
    for l in range(CHUNK // 16):
        ones_v[pl.ds(l * 16, 16)] = jnp.ones((16,), jnp.float32)
    plsc.subcore_barrier()

    def body(j, carry):
        pltpu.sync_copy(ones_v, acc.at[dst_v.at[j]], add=True)
        return carry

    lax.fori_loop(0, k_chunks, body, 0)
    plsc.subcore_barrier()
    pltpu.sync_copy(acc.at[pl.ds(base, rows)],
                    out_hbm.at[pl.ds(out_base, rows)])


DEPTH = 2


def _sc_agg_body(y_hbm, src_hbm, dst_hbm, zeros_hbm, out_hbm,
                 sb0, sb1, sb2, sb3, db0, db1, db2, db3,
                 r0, r1, r2, r3, acc,
                 ss0, ss1, ss2, ss3, sd0, sd1, sd2, sd3,
                 g0, g1, g2, g3, c0, c1, c2, c3,
                 k_chunks, n_pad, d):
    sbuf = (sb0, sb1, sb2, sb3)
    dbuf = (db0, db1, db2, db3)
    rows_v = (r0, r1, r2, r3)
    ssem = (ss0, ss1, ss2, ss3)
    dsem = (sd0, sd1, sd2, sd3)
    gsem = (g0, g1, g2, g3)
    csem = (c0, c1, c2, c3)
    cid = lax.axis_index("c")
    sid = lax.axis_index("s")
    wid = sid * NC + cid
    rows = n_pad // NS
    base = pl.multiple_of(sid * rows, 8)
    pltpu.sync_copy(zeros_hbm.at[pl.ds(base, rows)], acc.at[pl.ds(base, rows)])
    plsc.subcore_barrier()

    def fire_sidx(j, q):
        pltpu.async_copy(src_hbm.at[wid, j], sbuf[q], ssem[q])

    def fire_didx(j, q):
        pltpu.async_copy(dst_hbm.at[wid, j], dbuf[q], dsem[q])

    def fire_gather(j, q):
        pltpu.make_async_copy(src_hbm.at[wid, j], sbuf[q], ssem[q]).wait()
        pltpu.async_copy(y_hbm.at[sbuf[q]], rows_v[q], gsem[q])

    def scat_wait(q):
        pltpu.make_async_copy(rows_v[q], acc.at[dbuf[q]], csem[q]).wait()

    for q in range(4):
        fire_sidx(q, q)
    fire_didx(0, 0)
    fire_didx(1, 1)
    fire_gather(0, 0)
    fire_gather(1, 1)

    def body(jj, carry):
        for i in range(4):
            j = jj * 4 + i
            s2 = (i + 2) % 4
            pltpu.make_async_copy(y_hbm.at[sbuf[i]], rows_v[i],
                                  gsem[i]).wait()

            @pl.when(j >= 2)
            def _():
                scat_wait(s2)

            @pl.when(j + 2 < k_chunks)
            def _():
                fire_gather(j + 2, s2)
                fire_didx(j + 2, s2)

            @pl.when(j + 4 < k_chunks)
            def _():
                fire_sidx(j + 4, i)

            pltpu.make_async_copy(dst_hbm.at[wid, j], dbuf[i],
                                  dsem[i]).wait()
            pltpu.async_copy(rows_v[i], acc.at[dbuf[i]], csem[i], add=True)
        return carry

    lax.fori_loop(0, k_chunks // 4, body, 0)
    scat_wait((k_chunks - 2) % 4)
    scat_wait((k_chunks - 1) % 4)
    plsc.subcore_barrier()
    pltpu.sync_copy(acc.at[pl.ds(base, rows)],
                    out_hbm.at[cid, pl.ds(base, rows)])


def _tc_mm_body(x_ref, w_ref, y_ref):
    y_ref[...] = jnp.dot(x_ref[...], w_ref[...],
                         preferred_element_type=jnp.float32)


def _tc_scale_body(xw_ref, d0_ref, d1_ref, y_ref, dinv_ref):
    deg = d0_ref[...] + d1_ref[...] + 1.0
    dinv = lax.rsqrt(deg)
    dinv_ref[...] = dinv
    y_ref[...] = xw_ref[...] * dinv


def _tc_mid_body(s0_ref, s1_ref, y_ref, dinv_ref, b_ref, w_ref, y2_ref):
    dinv = dinv_ref[...]
    h = jnp.maximum(
        dinv * (s0_ref[0] + s1_ref[0] + y_ref[...]) + b_ref[...], 0.0)
    y2_ref[...] = jnp.dot(h, w_ref[...],
                          preferred_element_type=jnp.float32) * dinv


def _tc_out_body(s0_ref, s1_ref, y_ref, dinv_ref, b_ref, o_ref):
    o_ref[...] = jnp.maximum(
        dinv_ref[...] * (s0_ref[0] + s1_ref[0] + y_ref[...])
        + b_ref[...], 0.0)


def kernel(x, edge_index, W0, b0, W1, b1):
    n, d_in = x.shape
    d_hid = W0.shape[1]
    e = edge_index.shape[1]
    n_pad = _pad_rows(n)
    n_pad_a = (n // 128 + 1) * 128
    k_chunks = -(-e // (NW * CHUNK))
    k_chunks = max(4, -(-k_chunks // 4) * 4)
    e_pad = k_chunks * NW * CHUNK
    pad = e_pad - e

    src = edge_index[0].astype(jnp.int32)
    dst = edge_index[1].astype(jnp.int32)
    if pad:
        fill = np.arange(pad, dtype=np.int32)
        src = jnp.concatenate([src, jnp.asarray(fill % n)])
        dst = jnp.concatenate([dst, jnp.asarray(n + fill % (n_pad_a - n))])
    src_slab = src.reshape(NW, k_chunks, CHUNK)
    dst_slab = dst.reshape(NW, k_chunks, CHUNK)
    zeros1 = jnp.zeros((n_pad,), jnp.float32)
    zeros2 = jnp.zeros((n_pad_a, d_hid), jnp.float32)

    deg_fn = pl.kernel(
        functools.partial(_sc_deg_body, k_chunks=k_chunks, n_pad=n_pad),
        out_type=jax.ShapeDtypeStruct((NC * n_pad,), jnp.float32),
        mesh=_sc_mesh(),
        scratch_types=[
            pltpu.VMEM((k_chunks, CHUNK), jnp.int32),
            pltpu.VMEM((CHUNK,), jnp.float32),
            pltpu.VMEM_SHARED((n_pad,), jnp.float32),
        ],
    )
    degp = deg_fn(dst_slab, zeros1).reshape(NC * n_pad, 1)

    agg_fn = pl.kernel(
        functools.partial(_sc_agg_body, k_chunks=k_chunks, n_pad=n_pad_a,
                          d=d_hid),
        out_type=jax.ShapeDtypeStruct((NC, n_pad_a, d_hid), jnp.float32),
        mesh=_sc_mesh(),
        scratch_types=[pltpu.VMEM((CHUNK,), jnp.int32)] * 8
        + [pltpu.VMEM((CHUNK, d_hid), jnp.float32)] * 4 + [
            pltpu.VMEM_SHARED((n_pad_a, d_hid), jnp.float32),
        ] + [pltpu.SemaphoreType.DMA] * 16,
    )

    bm = 1024
    grid = (-(-n // bm),)
    row_spec = pl.BlockSpec((bm, d_hid), lambda i: (i, 0))
    col_spec = pl.BlockSpec((bm, 1), lambda i: (i, 0))
    w_spec = pl.BlockSpec((d_in, d_hid), lambda i: (0, 0))
    b_spec = pl.BlockSpec((1, d_hid), lambda i: (0, 0))

    xw1 = pl.pallas_call(
        _tc_mm_body,
        grid=grid,
        in_specs=[pl.BlockSpec((bm, d_in), lambda i: (i, 0)), w_spec],
        out_specs=row_spec,
        out_shape=jax.ShapeDtypeStruct((n, d_hid), jnp.float32),
    )(x, W0)

    off = n_pad // bm
    d0_spec = pl.BlockSpec((bm, 1), lambda i: (i, 0))
    d1_spec = pl.BlockSpec((bm, 1), lambda i: (i + off, 0))
    y1, dinv = pl.pallas_call(
        _tc_scale_body,
        grid=grid,
        in_specs=[row_spec, d0_spec, d1_spec],
        out_specs=[row_spec, col_spec],
        out_shape=[jax.ShapeDtypeStruct((n, d_hid), jnp.float32),
                   jax.ShapeDtypeStruct((n, 1), jnp.float32)],
    )(xw1, degp, degp)

    s_spec0 = pl.BlockSpec((1, bm, d_hid), lambda i: (0, i, 0))
    s_spec1 = pl.BlockSpec((1, bm, d_hid), lambda i: (1, i, 0))

    s1 = agg_fn(y1, src_slab, dst_slab, zeros2)
    y2 = pl.pallas_call(
        _tc_mid_body,
        grid=grid,
        in_specs=[s_spec0, s_spec1, row_spec, col_spec, b_spec, w_spec],
        out_specs=row_spec,
        out_shape=jax.ShapeDtypeStruct((n, d_hid), jnp.float32),
    )(s1, s1, y1, dinv, b0[None, :], W1)

    s2 = agg_fn(y2, src_slab, dst_slab, zeros2)
    out = pl.pallas_call(
        _tc_out_body,
        grid=grid,
        in_specs=[s_spec0, s_spec1, row_spec, col_spec, b_spec],
        out_specs=row_spec,
        out_shape=jax.ShapeDtypeStruct((n, d_hid), jnp.float32),
    )(s2, s2, y2, dinv, b1[None, :])
    return out


# speedup vs baseline: 1.0921x; 1.0165x over previous
import functools

import numpy as np

import jax
import jax.numpy as jnp
from jax import lax
from jax.experimental import pallas as pl
from jax.experimental.pallas import tpu as pltpu
from jax.experimental.pallas import tpu_sc as plsc

NC = 2
NS = 16
NW = NC * NS
CHUNK = 64

def _sc_mesh():
    return plsc.VectorSubcoreMesh(core_axis_name="c", subcore_axis_name="s")


def _pad_rows(n):
    return (n // 2048 + 1) * 2048


def _sc_deg_body(dst_hbm, zeros_hbm, out_hbm, dst_v, ones_v, acc, k_chunks,
                 n_pad):
    cid = lax.axis_index("c")
    sid = lax.axis_index("s")
    wid = sid * NC + cid
    rows = n_pad // NS
    base = pl.multiple_of(sid * rows, 128)
    out_base = pl.multiple_of(cid * n_pad + sid * rows, 128)
    pltpu.sync_copy(zeros_hbm.at[pl.ds(base, rows)], acc.at[pl.ds(base, rows)])
    pltpu.sync_copy(dst_hbm.at[wid], dst_v)
    for l in range(CHUNK // 16):
        ones_v[pl.ds(l * 16, 16)] = jnp.ones((16,), jnp.float32)
    plsc.subcore_barrier()

    def body(j, carry):
        pltpu.sync_copy(ones_v, acc.at[dst_v.at[j]], add=True)
        return carry

    lax.fori_loop(0, k_chunks, body, 0)
    plsc.subcore_barrier()
    pltpu.sync_copy(acc.at[pl.ds(base, rows)],
                    out_hbm.at[pl.ds(out_base, rows)])


DEPTH = 2


def _sc_agg_body(y_hbm, src_hbm, dst_hbm, zeros_hbm, out_hbm,
                 sb0, sb1, sb2, sb3, db0, db1, db2, db3,
                 r0, r1, r2, r3, acc,
                 ss0, ss1, ss2, ss3, sd0, sd1, sd2, sd3,
                 g0, g1, g2, g3, c0, c1, c2, c3,
                 k_chunks, n_pad, d):
    sbuf = (sb0, sb1, sb2, sb3)
    dbuf = (db0, db1, db2, db3)
    rows_v = (r0, r1, r2, r3)
    ssem = (ss0, ss1, ss2, ss3)
    dsem = (sd0, sd1, sd2, sd3)
    gsem = (g0, g1, g2, g3)
    csem = (c0, c1, c2, c3)
    cid = lax.axis_index("c")
    sid = lax.axis_index("s")
    wid = sid * NC + cid
    rows = n_pad // NS
    base = pl.multiple_of(sid * rows, 8)
    pltpu.sync_copy(zeros_hbm.at[pl.ds(base, rows)], acc.at[pl.ds(base, rows)])
    plsc.subcore_barrier()

    def fire_sidx(j, q):
        pltpu.async_copy(src_hbm.at[wid, j], sbuf[q], ssem[q])

    def fire_didx(j, q):
        pltpu.async_copy(dst_hbm.at[wid, j], dbuf[q], dsem[q])

    def fire_gather(j, q):
        pltpu.make_async_copy(src_hbm.at[wid, j], sbuf[q], ssem[q]).wait()
        pltpu.async_copy(y_hbm.at[sbuf[q]], rows_v[q], gsem[q])

    def scat_wait(q):
        pltpu.make_async_copy(rows_v[q], acc.at[dbuf[q]], csem[q]).wait()

    for q in range(4):
        fire_sidx(q, q)
    fire_didx(0, 0)
    fire_didx(1, 1)
    fire_gather(0, 0)
    fire_gather(1, 1)

    def body(jj, carry):
        for i in range(4):
            j = jj * 4 + i
            s2 = (i + 2) % 4
            pltpu.make_async_copy(y_hbm.at[sbuf[i]], rows_v[i],
                                  gsem[i]).wait()

            @pl.when(j >= 2)
            def _():
                scat_wait(s2)

            @pl.when(j + 2 < k_chunks)
            def _():
                fire_gather(j + 2, s2)
                fire_didx(j + 2, s2)

            @pl.when(j + 4 < k_chunks)
            def _():
                fire_sidx(j + 4, i)

            pltpu.make_async_copy(dst_hbm.at[wid, j], dbuf[i],
                                  dsem[i]).wait()
            pltpu.async_copy(rows_v[i], acc.at[dbuf[i]], csem[i], add=True)
        return carry

    lax.fori_loop(0, k_chunks // 4, body, 0)
    scat_wait((k_chunks - 2) % 4)
    scat_wait((k_chunks - 1) % 4)
    plsc.subcore_barrier()
    pltpu.sync_copy(acc.at[pl.ds(base, rows)],
                    out_hbm.at[cid, pl.ds(base, rows)])


def _tc_mm_body(x_ref, w_ref, y_ref):
    y_ref[...] = jnp.dot(x_ref[...], w_ref[...],
                         preferred_element_type=jnp.float32)


def _tc_scale_body(xw_ref, d0_ref, d1_ref, y_ref, dinv_ref):
    deg = d0_ref[...] + d1_ref[...] + 1.0
    dinv = lax.rsqrt(deg)
    dinv_ref[...] = dinv
    y_ref[...] = xw_ref[...] * dinv


def _tc_mid_body(s0_ref, s1_ref, y_ref, dinv_ref, b_ref, w_ref, y2_ref):
    dinv = dinv_ref[...]
    h = jnp.maximum(
        dinv * (s0_ref[0] + s1_ref[0] + y_ref[...]) + b_ref[...], 0.0)
    y2_ref[...] = jnp.dot(h, w_ref[...],
                          preferred_element_type=jnp.float32) * dinv


def _tc_out_body(s0_ref, s1_ref, y_ref, dinv_ref, b_ref, o_ref):
    o_ref[...] = jnp.maximum(
        dinv_ref[...] * (s0_ref[0] + s1_ref[0] + y_ref[...])
        + b_ref[...], 0.0)


def kernel(x, edge_index, W0, b0, W1, b1):
    n, d_in = x.shape
    d_hid = W0.shape[1]
    e = edge_index.shape[1]
    n_pad = _pad_rows(n)
    n_pad_a = (n // 128 + 1) * 128
    k_chunks = -(-e // (NW * CHUNK))
    k_chunks = max(4, -(-k_chunks // 4) * 4)
    e_pad = k_chunks * NW * CHUNK
    pad = e_pad - e

    src = edge_index[0].astype(jnp.int32)
    dst = edge_index[1].astype(jnp.int32)
    if pad:
        fill = np.arange(pad, dtype=np.int32)
        src = jnp.concatenate([src, jnp.asarray(fill % n)])
        dst = jnp.concatenate([dst, jnp.asarray(n + fill % (n_pad_a - n))])
    src_slab = src.reshape(NW, k_chunks, CHUNK)
    dst_slab = dst.reshape(NW, k_chunks, CHUNK)
    zeros1 = jnp.zeros((n_pad,), jnp.float32)
    zeros2 = jnp.zeros((n_pad_a, d_hid), jnp.float32)

    deg_fn = pl.kernel(
        functools.partial(_sc_deg_body, k_chunks=k_chunks, n_pad=n_pad),
        out_type=jax.ShapeDtypeStruct((NC * n_pad,), jnp.float32),
        mesh=_sc_mesh(),
        scratch_types=[
            pltpu.VMEM((k_chunks, CHUNK), jnp.int32),
            pltpu.VMEM((CHUNK,), jnp.float32),
            pltpu.VMEM_SHARED((n_pad,), jnp.float32),
        ],
    )
    degp = deg_fn(dst_slab, zeros1).reshape(NC * n_pad, 1)

    agg_fn = pl.kernel(
        functools.partial(_sc_agg_body, k_chunks=k_chunks, n_pad=n_pad_a,
                          d=d_hid),
        out_type=jax.ShapeDtypeStruct((NC, n_pad_a, d_hid), jnp.float32),
        mesh=_sc_mesh(),
        scratch_types=[pltpu.VMEM((CHUNK,), jnp.int32)] * 8
        + [pltpu.VMEM((CHUNK, d_hid), jnp.float32)] * 4 + [
            pltpu.VMEM_SHARED((n_pad_a, d_hid), jnp.float32),
        ] + [pltpu.SemaphoreType.DMA] * 16,
    )

    bm = 2048
    grid = (-(-n // bm),)
    row_spec = pl.BlockSpec((bm, d_hid), lambda i: (i, 0))
    col_spec = pl.BlockSpec((bm, 1), lambda i: (i, 0))
    w_spec = pl.BlockSpec((d_in, d_hid), lambda i: (0, 0))
    b_spec = pl.BlockSpec((1, d_hid), lambda i: (0, 0))

    xw1 = pl.pallas_call(
        _tc_mm_body,
        grid=grid,
        in_specs=[pl.BlockSpec((bm, d_in), lambda i: (i, 0)), w_spec],
        out_specs=row_spec,
        out_shape=jax.ShapeDtypeStruct((n, d_hid), jnp.float32),
    )(x, W0)

    off = n_pad // bm
    d0_spec = pl.BlockSpec((bm, 1), lambda i: (i, 0))
    d1_spec = pl.BlockSpec((bm, 1), lambda i: (i + off, 0))
    y1, dinv = pl.pallas_call(
        _tc_scale_body,
        grid=grid,
        in_specs=[row_spec, d0_spec, d1_spec],
        out_specs=[row_spec, col_spec],
        out_shape=[jax.ShapeDtypeStruct((n, d_hid), jnp.float32),
                   jax.ShapeDtypeStruct((n, 1), jnp.float32)],
    )(xw1, degp, degp)

    s_spec0 = pl.BlockSpec((1, bm, d_hid), lambda i: (0, i, 0))
    s_spec1 = pl.BlockSpec((1, bm, d_hid), lambda i: (1, i, 0))

    s1 = agg_fn(y1, src_slab, dst_slab, zeros2)
    y2 = pl.pallas_call(
        _tc_mid_body,
        grid=grid,
        in_specs=[s_spec0, s_spec1, row_spec, col_spec, b_spec, w_spec],
        out_specs=row_spec,
        out_shape=jax.ShapeDtypeStruct((n, d_hid), jnp.float32),
    )(s1, s1, y1, dinv, b0[None, :], W1)

    s2 = agg_fn(y2, src_slab, dst_slab, zeros2)
    out = pl.pallas_call(
        _tc_out_body,
        grid=grid,
        in_specs=[s_spec0, s_spec1, row_spec, col_spec, b_spec],
        out_specs=row_spec,
        out_shape=jax.ShapeDtypeStruct((n, d_hid), jnp.float32),
    )(s2, s2, y2, dinv, b1[None, :])
    return out
